# Initial kernel scaffold; baseline (speedup 1.0000x reference)
import jax
import jax.numpy as jnp
from jax.experimental import pallas as pl


def kernel(features, edge_index, W1, b1, W2, b2):
    raise NotImplementedError("write your pallas kernel here")


# speedup vs baseline: 7.8718x; 7.8718x over previous
import functools

import jax
import jax.numpy as jnp
from jax import lax
from jax.experimental import pallas as pl
from jax.experimental.pallas import tpu as pltpu
from jax.experimental.pallas import tpu_sc as plsc

N = 10000
E = 320000
IN_DIM = 128
HID_DIM = 256
OUT_DIM = 128

NC = 2
NS = 16
NW = NC * NS
RPT = 624
TAIL0 = RPT * NS
TAIL = N - TAIL0
GROUP = 100
N_GROUPS = E // GROUP
GROUPS_PER_TILE = N_GROUPS // NW

_mesh = plsc.VectorSubcoreMesh(
    core_axis_name="c", subcore_axis_name="s", num_cores=NC, num_subcores=NS)


def _worker(c, s):
    return s * NC + c


def _zero_stripe(s, zeros_hbm, sh):
    pltpu.sync_copy(zeros_hbm.at[pl.ds(s * RPT, RPT)],
                    sh.at[pl.ds(s * RPT, RPT)])

    @pl.when(s == NS - 1)
    def _():
        pltpu.sync_copy(zeros_hbm.at[pl.ds(TAIL0, TAIL)],
                        sh.at[pl.ds(TAIL0, TAIL)])


def _publish_stripe(c, s, sh, out_hbm):
    pltpu.sync_copy(sh.at[pl.ds(s * RPT, RPT)],
                    out_hbm.at[c, pl.ds(s * RPT, RPT)])

    @pl.when(s == NS - 1)
    def _():
        pltpu.sync_copy(sh.at[pl.ds(TAIL0, TAIL)],
                        out_hbm.at[c, pl.ds(TAIL0, TAIL)])


def _sc_degrees_body(src_g_hbm, dst_g_hbm, ones_hbm, zeros_hbm, deg_hbm,
                     src_v, dst_v, ones_v, acc_sh):
    c = lax.axis_index("c")
    s = lax.axis_index("s")
    _zero_stripe(s, zeros_hbm, acc_sh)
    w = _worker(c, s)
    pltpu.sync_copy(src_g_hbm.at[w], src_v)
    pltpu.sync_copy(dst_g_hbm.at[w], dst_v)
    pltpu.sync_copy(ones_hbm.at[0], ones_v)
    plsc.subcore_barrier()

    def body0(g, carry):
        pltpu.sync_copy(ones_v, acc_sh.at[src_v.at[g]], add=True)
        return carry

    lax.fori_loop(0, GROUPS_PER_TILE, body0, 0)
    pltpu.sync_copy(ones_hbm.at[1], ones_v)

    def body1(g, carry):
        pltpu.sync_copy(ones_v, acc_sh.at[dst_v.at[g]], add=True)
        return carry

    lax.fori_loop(0, GROUPS_PER_TILE, body1, 0)
    plsc.subcore_barrier()
    _publish_stripe(c, s, acc_sh, deg_hbm)


_sc_degrees = pl.kernel(
    _sc_degrees_body,
    out_type=jax.ShapeDtypeStruct((NC, N, IN_DIM), jnp.float32),
    mesh=_mesh,
    scratch_types=[
        pltpu.VMEM((GROUPS_PER_TILE, GROUP), jnp.int32),
        pltpu.VMEM((GROUPS_PER_TILE, GROUP), jnp.int32),
        pltpu.VMEM((GROUP, IN_DIM), jnp.float32),
        pltpu.VMEM_SHARED((N, IN_DIM), jnp.float32),
    ],
)


def _sc_aggregate_body(y_hbm, src_g_hbm, dst_g_hbm, zeros_hbm, out_hbm,
                       src_v, dst_v, rows_v, acc_sh, sem):
    c = lax.axis_index("c")
    s = lax.axis_index("s")
    _zero_stripe(s, zeros_hbm, acc_sh)
    w = _worker(c, s)
    pltpu.sync_copy(src_g_hbm.at[w], src_v)
    pltpu.sync_copy(dst_g_hbm.at[w], dst_v)
    plsc.subcore_barrier()

    def body(g, carry):
        pltpu.async_copy(y_hbm.at[src_v.at[g]], rows_v, sem).wait()
        pltpu.sync_copy(rows_v, acc_sh.at[dst_v.at[g]], add=True)
        return carry

    lax.fori_loop(0, GROUPS_PER_TILE, body, 0)
    plsc.subcore_barrier()
    _publish_stripe(c, s, acc_sh, out_hbm)


_sc_aggregate = pl.kernel(
    _sc_aggregate_body,
    out_type=jax.ShapeDtypeStruct((NC, N, IN_DIM), jnp.float32),
    mesh=_mesh,
    scratch_types=[
        pltpu.VMEM((GROUPS_PER_TILE, GROUP), jnp.int32),
        pltpu.VMEM((GROUPS_PER_TILE, GROUP), jnp.int32),
        pltpu.VMEM((GROUP, IN_DIM), jnp.float32),
        pltpu.VMEM_SHARED((N, IN_DIM), jnp.float32),
        pltpu.SemaphoreType.DMA,
    ],
)


RB = 1000


def _deg_scale(deg_ref, col):
    deg = deg_ref[0, :, col] + deg_ref[1, :, col]
    return lax.rsqrt(jnp.maximum(deg, 1.0))


def _tc_prep_body(deg_ref, x_ref, xs_ref):
    xs_ref[...] = x_ref[...] * _deg_scale(deg_ref, 0)[:, None]


def _tc_prep(deg_p, x):
    return pl.pallas_call(
        _tc_prep_body,
        grid=(N // RB,),
        in_specs=[
            pl.BlockSpec((NC, RB, IN_DIM), lambda i: (0, i, 0)),
            pl.BlockSpec((RB, IN_DIM), lambda i: (i, 0)),
        ],
        out_specs=pl.BlockSpec((RB, IN_DIM), lambda i: (i, 0)),
        out_shape=jax.ShapeDtypeStruct((N, IN_DIM), jnp.float32),
    )(deg_p, x)


def _tc_dense_body(aggp_ref, deg_ref, w1_ref, b1_ref, w2_ref, h2_ref):
    agg = aggp_ref[0] + aggp_ref[1]
    si = _deg_scale(deg_ref, 1)
    so = _deg_scale(deg_ref, 0)
    t = jnp.dot(agg, w1_ref[...], preferred_element_type=jnp.float32)
    t = t * si[:, None] + b1_ref[...]
    t = jnp.maximum(t, 0.0) * so[:, None]
    h2_ref[...] = jnp.dot(t, w2_ref[...], preferred_element_type=jnp.float32)


def _tc_dense(agg1_p, deg_p, w1, b1, w2):
    return pl.pallas_call(
        _tc_dense_body,
        grid=(N // RB,),
        in_specs=[
            pl.BlockSpec((NC, RB, IN_DIM), lambda i: (0, i, 0)),
            pl.BlockSpec((NC, RB, IN_DIM), lambda i: (0, i, 0)),
            pl.BlockSpec((IN_DIM, HID_DIM), lambda i: (0, 0)),
            pl.BlockSpec((1, HID_DIM), lambda i: (0, 0)),
            pl.BlockSpec((HID_DIM, OUT_DIM), lambda i: (0, 0)),
        ],
        out_specs=pl.BlockSpec((RB, OUT_DIM), lambda i: (i, 0)),
        out_shape=jax.ShapeDtypeStruct((N, OUT_DIM), jnp.float32),
    )(agg1_p, deg_p, w1, b1, w2)


def _tc_final_body(aggp_ref, deg_ref, b2_ref, out_ref):
    agg = aggp_ref[0] + aggp_ref[1]
    si = _deg_scale(deg_ref, 1)
    out_ref[...] = agg * si[:, None] + b2_ref[...]


def _tc_final(agg2_p, deg_p, b2):
    return pl.pallas_call(
        _tc_final_body,
        grid=(N // RB,),
        in_specs=[
            pl.BlockSpec((NC, RB, OUT_DIM), lambda i: (0, i, 0)),
            pl.BlockSpec((NC, RB, IN_DIM), lambda i: (0, i, 0)),
            pl.BlockSpec((1, OUT_DIM), lambda i: (0, 0)),
        ],
        out_specs=pl.BlockSpec((RB, OUT_DIM), lambda i: (i, 0)),
        out_shape=jax.ShapeDtypeStruct((N, OUT_DIM), jnp.float32),
    )(agg2_p, deg_p, b2)


def kernel(features, edge_index, W1, b1, W2, b2):
    src_g = edge_index[0].reshape(NW, GROUPS_PER_TILE, GROUP)
    dst_g = edge_index[1].reshape(NW, GROUPS_PER_TILE, GROUP)
    ones2 = jnp.zeros((2, GROUP, IN_DIM), jnp.float32)
    ones2 = ones2.at[0, :, 0].set(1.0).at[1, :, 1].set(1.0)
    zeros128 = jnp.zeros((N, IN_DIM), jnp.float32)

    deg_p = _sc_degrees(src_g, dst_g, ones2, zeros128)
    x_scaled = _tc_prep(deg_p, features)
    agg1_p = _sc_aggregate(x_scaled, src_g, dst_g, zeros128)
    h2 = _tc_dense(agg1_p, deg_p, W1, b1.reshape(1, HID_DIM), W2)
    agg2_p = _sc_aggregate(h2, src_g, dst_g, zeros128)
    return _tc_final(agg2_p, deg_p, b2.reshape(1, OUT_DIM))


# speedup vs baseline: 9.3779x; 1.1913x over previous
import functools

import jax
import jax.numpy as jnp
from jax import lax
from jax.experimental import pallas as pl
from jax.experimental.pallas import tpu as pltpu
from jax.experimental.pallas import tpu_sc as plsc

N = 10000
E = 320000
IN_DIM = 128
HID_DIM = 256
OUT_DIM = 128

NC = 2
NS = 16
NW = NC * NS
RPT = 624
TAIL0 = RPT * NS
TAIL = N - TAIL0
GROUP = 100
N_GROUPS = E // GROUP
GROUPS_PER_TILE = N_GROUPS // NW
STAGE_A = 56
STAGE_B = GROUPS_PER_TILE - STAGE_A

_mesh = plsc.VectorSubcoreMesh(
    core_axis_name="c", subcore_axis_name="s", num_cores=NC, num_subcores=NS)


def _worker(c, s):
    return s * NC + c


def _zero_stripe(s, zeros_hbm, sh):
    pltpu.sync_copy(zeros_hbm.at[pl.ds(s * RPT, RPT)],
                    sh.at[pl.ds(s * RPT, RPT)])

    @pl.when(s == NS - 1)
    def _():
        pltpu.sync_copy(zeros_hbm.at[pl.ds(TAIL0, TAIL)],
                        sh.at[pl.ds(TAIL0, TAIL)])


def _publish_stripe(c, s, sh, out_hbm):
    pltpu.sync_copy(sh.at[pl.ds(s * RPT, RPT)],
                    out_hbm.at[c, pl.ds(s * RPT, RPT)])

    @pl.when(s == NS - 1)
    def _():
        pltpu.sync_copy(sh.at[pl.ds(TAIL0, TAIL)],
                        out_hbm.at[c, pl.ds(TAIL0, TAIL)])


def _sc_degrees_body(src_g_hbm, dst_g_hbm, ones_hbm, zeros_hbm, deg_hbm,
                     src_v, dst_v, ones_v, acc_sh, sem):
    c = lax.axis_index("c")
    s = lax.axis_index("s")
    _zero_stripe(s, zeros_hbm, acc_sh)
    w = _worker(c, s)
    pltpu.sync_copy(src_g_hbm.at[w], src_v)
    pltpu.sync_copy(dst_g_hbm.at[w], dst_v)
    plsc.subcore_barrier()

    K = 10

    def scatter_pass(idx_v):
        def chunk(k, carry):
            for j in range(K):
                pltpu.async_copy(ones_v, acc_sh.at[idx_v.at[k * K + j]],
                                 sem, add=True)
            for j in range(K):
                pltpu.make_async_copy(ones_v, acc_sh.at[idx_v.at[k * K + j]],
                                      sem).wait()
            return carry

        lax.fori_loop(0, GROUPS_PER_TILE // K, chunk, 0)

    pltpu.sync_copy(ones_hbm.at[0], ones_v)
    scatter_pass(src_v)
    pltpu.sync_copy(ones_hbm.at[1], ones_v)
    scatter_pass(dst_v)
    plsc.subcore_barrier()
    _publish_stripe(c, s, acc_sh, deg_hbm)


_sc_degrees = pl.kernel(
    _sc_degrees_body,
    out_type=jax.ShapeDtypeStruct((NC, N, IN_DIM), jnp.float32),
    mesh=_mesh,
    scratch_types=[
        pltpu.VMEM((GROUPS_PER_TILE, GROUP), jnp.int32),
        pltpu.VMEM((GROUPS_PER_TILE, GROUP), jnp.int32),
        pltpu.VMEM((GROUP, IN_DIM), jnp.float32),
        pltpu.VMEM_SHARED((N, IN_DIM), jnp.float32),
        pltpu.SemaphoreType.DMA,
    ],
)


def _sc_aggregate_body(y_hbm, src_g_hbm, dst_g_hbm, zeros_hbm, out_hbm,
                       src_v, dst_v, rows_a, rows_b, acc_sh, sem_a, sem_b):
    c = lax.axis_index("c")
    s = lax.axis_index("s")
    _zero_stripe(s, zeros_hbm, acc_sh)
    w = _worker(c, s)

    def stage(off, n):
        pltpu.sync_copy(src_g_hbm.at[w, pl.ds(off, n)], src_v.at[pl.ds(0, n)])
        pltpu.sync_copy(dst_g_hbm.at[w, pl.ds(off, n)], dst_v.at[pl.ds(0, n)])

    def span(n):
        pltpu.async_copy(y_hbm.at[src_v.at[0]], rows_a, sem_a)

        def body(i, carry):
            ga = 2 * i
            gb = 2 * i + 1
            pltpu.make_async_copy(y_hbm.at[src_v.at[ga]], rows_a, sem_a).wait()
            pltpu.async_copy(y_hbm.at[src_v.at[gb]], rows_b, sem_b)
            pltpu.sync_copy(rows_a, acc_sh.at[dst_v.at[ga]], add=True)
            pltpu.make_async_copy(y_hbm.at[src_v.at[gb]], rows_b, sem_b).wait()

            @pl.when(i < n // 2 - 1)
            def _():
                pltpu.async_copy(y_hbm.at[src_v.at[gb + 1]], rows_a, sem_a)

            pltpu.sync_copy(rows_b, acc_sh.at[dst_v.at[gb]], add=True)
            return carry

        lax.fori_loop(0, n // 2, body, 0)

    stage(0, STAGE_A)
    plsc.subcore_barrier()
    span(STAGE_A)
    stage(STAGE_A, STAGE_B)
    span(STAGE_B)
    plsc.subcore_barrier()
    _publish_stripe(c, s, acc_sh, out_hbm)


_sc_aggregate = pl.kernel(
    _sc_aggregate_body,
    out_type=jax.ShapeDtypeStruct((NC, N, IN_DIM), jnp.float32),
    mesh=_mesh,
    scratch_types=[
        pltpu.VMEM((STAGE_A, GROUP), jnp.int32),
        pltpu.VMEM((STAGE_A, GROUP), jnp.int32),
        pltpu.VMEM((GROUP, IN_DIM), jnp.float32),
        pltpu.VMEM((GROUP, IN_DIM), jnp.float32),
        pltpu.VMEM_SHARED((N, IN_DIM), jnp.float32),
        pltpu.SemaphoreType.DMA,
        pltpu.SemaphoreType.DMA,
    ],
)


RB = 1000


def _deg_scale(deg_ref, col):
    deg = deg_ref[0, :, col] + deg_ref[1, :, col]
    return lax.rsqrt(jnp.maximum(deg, 1.0))


def _tc_prep_body(deg_ref, x_ref, xs_ref):
    xs_ref[...] = x_ref[...] * _deg_scale(deg_ref, 0)[:, None]


def _tc_prep(deg_p, x):
    return pl.pallas_call(
        _tc_prep_body,
        grid=(N // RB,),
        in_specs=[
            pl.BlockSpec((NC, RB, IN_DIM), lambda i: (0, i, 0)),
            pl.BlockSpec((RB, IN_DIM), lambda i: (i, 0)),
        ],
        out_specs=pl.BlockSpec((RB, IN_DIM), lambda i: (i, 0)),
        out_shape=jax.ShapeDtypeStruct((N, IN_DIM), jnp.float32),
    )(deg_p, x)


def _tc_dense_body(aggp_ref, deg_ref, w1_ref, b1_ref, w2_ref, h2_ref):
    agg = aggp_ref[0] + aggp_ref[1]
    si = _deg_scale(deg_ref, 1)
    so = _deg_scale(deg_ref, 0)
    t = jnp.dot(agg, w1_ref[...], preferred_element_type=jnp.float32)
    t = t * si[:, None] + b1_ref[...]
    t = jnp.maximum(t, 0.0) * so[:, None]
    h2_ref[...] = jnp.dot(t, w2_ref[...], preferred_element_type=jnp.float32)


def _tc_dense(agg1_p, deg_p, w1, b1, w2):
    return pl.pallas_call(
        _tc_dense_body,
        grid=(N // RB,),
        in_specs=[
            pl.BlockSpec((NC, RB, IN_DIM), lambda i: (0, i, 0)),
            pl.BlockSpec((NC, RB, IN_DIM), lambda i: (0, i, 0)),
            pl.BlockSpec((IN_DIM, HID_DIM), lambda i: (0, 0)),
            pl.BlockSpec((1, HID_DIM), lambda i: (0, 0)),
            pl.BlockSpec((HID_DIM, OUT_DIM), lambda i: (0, 0)),
        ],
        out_specs=pl.BlockSpec((RB, OUT_DIM), lambda i: (i, 0)),
        out_shape=jax.ShapeDtypeStruct((N, OUT_DIM), jnp.float32),
    )(agg1_p, deg_p, w1, b1, w2)


def _tc_final_body(aggp_ref, deg_ref, b2_ref, out_ref):
    agg = aggp_ref[0] + aggp_ref[1]
    si = _deg_scale(deg_ref, 1)
    out_ref[...] = agg * si[:, None] + b2_ref[...]


def _tc_final(agg2_p, deg_p, b2):
    return pl.pallas_call(
        _tc_final_body,
        grid=(N // RB,),
        in_specs=[
            pl.BlockSpec((NC, RB, OUT_DIM), lambda i: (0, i, 0)),
            pl.BlockSpec((NC, RB, IN_DIM), lambda i: (0, i, 0)),
            pl.BlockSpec((1, OUT_DIM), lambda i: (0, 0)),
        ],
        out_specs=pl.BlockSpec((RB, OUT_DIM), lambda i: (i, 0)),
        out_shape=jax.ShapeDtypeStruct((N, OUT_DIM), jnp.float32),
    )(agg2_p, deg_p, b2)


def kernel(features, edge_index, W1, b1, W2, b2):
    src_g = edge_index[0].reshape(NW, GROUPS_PER_TILE, GROUP)
    dst_g = edge_index[1].reshape(NW, GROUPS_PER_TILE, GROUP)
    ones2 = jnp.zeros((2, GROUP, IN_DIM), jnp.float32)
    ones2 = ones2.at[0, :, 0].set(1.0).at[1, :, 1].set(1.0)
    zeros128 = jnp.zeros((N, IN_DIM), jnp.float32)

    deg_p = _sc_degrees(src_g, dst_g, ones2, zeros128)
    x_scaled = _tc_prep(deg_p, features)
    agg1_p = _sc_aggregate(x_scaled, src_g, dst_g, zeros128)
    h2 = _tc_dense(agg1_p, deg_p, W1, b1.reshape(1, HID_DIM), W2)
    agg2_p = _sc_aggregate(h2, src_g, dst_g, zeros128)
    return _tc_final(agg2_p, deg_p, b2.reshape(1, OUT_DIM))


# speedup vs baseline: 9.7870x; 1.0436x over previous
import functools

import jax
import jax.numpy as jnp
from jax import lax
from jax.experimental import pallas as pl
from jax.experimental.pallas import tpu as pltpu
from jax.experimental.pallas import tpu_sc as plsc

N = 10000
E = 320000
IN_DIM = 128
HID_DIM = 256
OUT_DIM = 128

NC = 2
NS = 16
NW = NC * NS
RPT = 624
TAIL0 = RPT * NS
TAIL = N - TAIL0
GROUP = 125
N_GROUPS = E // GROUP
GROUPS_PER_TILE = N_GROUPS // NW
STAGE_A = 40
STAGE_B = GROUPS_PER_TILE - STAGE_A

_mesh = plsc.VectorSubcoreMesh(
    core_axis_name="c", subcore_axis_name="s", num_cores=NC, num_subcores=NS)


def _worker(c, s):
    return s * NC + c


def _zero_stripe(s, zeros_hbm, sh):
    pltpu.sync_copy(zeros_hbm.at[pl.ds(s * RPT, RPT)],
                    sh.at[pl.ds(s * RPT, RPT)])

    @pl.when(s == NS - 1)
    def _():
        pltpu.sync_copy(zeros_hbm.at[pl.ds(TAIL0, TAIL)],
                        sh.at[pl.ds(TAIL0, TAIL)])


def _publish_stripe(c, s, sh, out_hbm):
    pltpu.sync_copy(sh.at[pl.ds(s * RPT, RPT)],
                    out_hbm.at[c, pl.ds(s * RPT, RPT)])

    @pl.when(s == NS - 1)
    def _():
        pltpu.sync_copy(sh.at[pl.ds(TAIL0, TAIL)],
                        out_hbm.at[c, pl.ds(TAIL0, TAIL)])


def _sc_degrees_body(src_g_hbm, dst_g_hbm, ones_hbm, zeros_hbm, deg_hbm,
                     src_v, dst_v, ones_v, acc_sh, sem):
    c = lax.axis_index("c")
    s = lax.axis_index("s")
    _zero_stripe(s, zeros_hbm, acc_sh)
    w = _worker(c, s)
    pltpu.sync_copy(src_g_hbm.at[w], src_v)
    pltpu.sync_copy(dst_g_hbm.at[w], dst_v)
    plsc.subcore_barrier()

    K = 10

    def scatter_pass(idx_v):
        def chunk(k, carry):
            for j in range(K):
                pltpu.async_copy(ones_v, acc_sh.at[idx_v.at[k * K + j]],
                                 sem, add=True)
            for j in range(K):
                pltpu.make_async_copy(ones_v, acc_sh.at[idx_v.at[k * K + j]],
                                      sem).wait()
            return carry

        lax.fori_loop(0, GROUPS_PER_TILE // K, chunk, 0)

    pltpu.sync_copy(ones_hbm.at[0], ones_v)
    scatter_pass(src_v)
    pltpu.sync_copy(ones_hbm.at[1], ones_v)
    scatter_pass(dst_v)
    plsc.subcore_barrier()
    _publish_stripe(c, s, acc_sh, deg_hbm)


_sc_degrees = pl.kernel(
    _sc_degrees_body,
    out_type=jax.ShapeDtypeStruct((NC, N, IN_DIM), jnp.float32),
    mesh=_mesh,
    scratch_types=[
        pltpu.VMEM((GROUPS_PER_TILE, GROUP), jnp.int32),
        pltpu.VMEM((GROUPS_PER_TILE, GROUP), jnp.int32),
        pltpu.VMEM((GROUP, IN_DIM), jnp.float32),
        pltpu.VMEM_SHARED((N, IN_DIM), jnp.float32),
        pltpu.SemaphoreType.DMA,
    ],
)


def _sc_aggregate_body(y_hbm, src_g_hbm, dst_g_hbm, zeros_hbm, out_hbm,
                       src_v, dst_v, rows_a, rows_b, acc_sh, sem_a, sem_b):
    c = lax.axis_index("c")
    s = lax.axis_index("s")
    _zero_stripe(s, zeros_hbm, acc_sh)
    w = _worker(c, s)

    def stage(off, n):
        pltpu.sync_copy(src_g_hbm.at[w, pl.ds(off, n)], src_v.at[pl.ds(0, n)])
        pltpu.sync_copy(dst_g_hbm.at[w, pl.ds(off, n)], dst_v.at[pl.ds(0, n)])

    def span(n):
        pltpu.async_copy(y_hbm.at[src_v.at[0]], rows_a, sem_a)

        def body(i, carry):
            ga = 2 * i
            gb = 2 * i + 1
            pltpu.make_async_copy(y_hbm.at[src_v.at[ga]], rows_a, sem_a).wait()
            pltpu.async_copy(y_hbm.at[src_v.at[gb]], rows_b, sem_b)
            pltpu.sync_copy(rows_a, acc_sh.at[dst_v.at[ga]], add=True)
            pltpu.make_async_copy(y_hbm.at[src_v.at[gb]], rows_b, sem_b).wait()

            @pl.when(i < n // 2 - 1)
            def _():
                pltpu.async_copy(y_hbm.at[src_v.at[gb + 1]], rows_a, sem_a)

            pltpu.sync_copy(rows_b, acc_sh.at[dst_v.at[gb]], add=True)
            return carry

        lax.fori_loop(0, n // 2, body, 0)

    stage(0, STAGE_A)
    plsc.subcore_barrier()
    span(STAGE_A)
    stage(STAGE_A, STAGE_B)
    span(STAGE_B)
    plsc.subcore_barrier()
    _publish_stripe(c, s, acc_sh, out_hbm)


_sc_aggregate = pl.kernel(
    _sc_aggregate_body,
    out_type=jax.ShapeDtypeStruct((NC, N, IN_DIM), jnp.float32),
    mesh=_mesh,
    scratch_types=[
        pltpu.VMEM((STAGE_A, GROUP), jnp.int32),
        pltpu.VMEM((STAGE_A, GROUP), jnp.int32),
        pltpu.VMEM((GROUP, IN_DIM), jnp.float32),
        pltpu.VMEM((GROUP, IN_DIM), jnp.float32),
        pltpu.VMEM_SHARED((N, IN_DIM), jnp.float32),
        pltpu.SemaphoreType.DMA,
        pltpu.SemaphoreType.DMA,
    ],
)


RB = 1000


def _deg_scale(deg_ref, col):
    deg = deg_ref[0, :, col] + deg_ref[1, :, col]
    return lax.rsqrt(jnp.maximum(deg, 1.0))


def _tc_prep_body(deg_ref, x_ref, xs_ref):
    xs_ref[...] = x_ref[...] * _deg_scale(deg_ref, 0)[:, None]


def _tc_prep(deg_p, x):
    return pl.pallas_call(
        _tc_prep_body,
        grid=(N // RB,),
        in_specs=[
            pl.BlockSpec((NC, RB, IN_DIM), lambda i: (0, i, 0)),
            pl.BlockSpec((RB, IN_DIM), lambda i: (i, 0)),
        ],
        out_specs=pl.BlockSpec((RB, IN_DIM), lambda i: (i, 0)),
        out_shape=jax.ShapeDtypeStruct((N, IN_DIM), jnp.float32),
    )(deg_p, x)


def _tc_dense_body(aggp_ref, deg_ref, w1_ref, b1_ref, w2_ref, h2_ref):
    agg = aggp_ref[0] + aggp_ref[1]
    si = _deg_scale(deg_ref, 1)
    so = _deg_scale(deg_ref, 0)
    t = jnp.dot(agg, w1_ref[...], preferred_element_type=jnp.float32)
    t = t * si[:, None] + b1_ref[...]
    t = jnp.maximum(t, 0.0) * so[:, None]
    h2_ref[...] = jnp.dot(t, w2_ref[...], preferred_element_type=jnp.float32)


def _tc_dense(agg1_p, deg_p, w1, b1, w2):
    return pl.pallas_call(
        _tc_dense_body,
        grid=(N // RB,),
        in_specs=[
            pl.BlockSpec((NC, RB, IN_DIM), lambda i: (0, i, 0)),
            pl.BlockSpec((NC, RB, IN_DIM), lambda i: (0, i, 0)),
            pl.BlockSpec((IN_DIM, HID_DIM), lambda i: (0, 0)),
            pl.BlockSpec((1, HID_DIM), lambda i: (0, 0)),
            pl.BlockSpec((HID_DIM, OUT_DIM), lambda i: (0, 0)),
        ],
        out_specs=pl.BlockSpec((RB, OUT_DIM), lambda i: (i, 0)),
        out_shape=jax.ShapeDtypeStruct((N, OUT_DIM), jnp.float32),
    )(agg1_p, deg_p, w1, b1, w2)


def _tc_final_body(aggp_ref, deg_ref, b2_ref, out_ref):
    agg = aggp_ref[0] + aggp_ref[1]
    si = _deg_scale(deg_ref, 1)
    out_ref[...] = agg * si[:, None] + b2_ref[...]


def _tc_final(agg2_p, deg_p, b2):
    return pl.pallas_call(
        _tc_final_body,
        grid=(N // RB,),
        in_specs=[
            pl.BlockSpec((NC, RB, OUT_DIM), lambda i: (0, i, 0)),
            pl.BlockSpec((NC, RB, IN_DIM), lambda i: (0, i, 0)),
            pl.BlockSpec((1, OUT_DIM), lambda i: (0, 0)),
        ],
        out_specs=pl.BlockSpec((RB, OUT_DIM), lambda i: (i, 0)),
        out_shape=jax.ShapeDtypeStruct((N, OUT_DIM), jnp.float32),
    )(agg2_p, deg_p, b2)


def kernel(features, edge_index, W1, b1, W2, b2):
    src_g = edge_index[0].reshape(NW, GROUPS_PER_TILE, GROUP)
    dst_g = edge_index[1].reshape(NW, GROUPS_PER_TILE, GROUP)
    ones2 = jnp.zeros((2, GROUP, IN_DIM), jnp.float32)
    ones2 = ones2.at[0, :, 0].set(1.0).at[1, :, 1].set(1.0)
    zeros128 = jnp.zeros((N, IN_DIM), jnp.float32)

    deg_p = _sc_degrees(src_g, dst_g, ones2, zeros128)
    x_scaled = _tc_prep(deg_p, features)
    agg1_p = _sc_aggregate(x_scaled, src_g, dst_g, zeros128)
    h2 = _tc_dense(agg1_p, deg_p, W1, b1.reshape(1, HID_DIM), W2)
    agg2_p = _sc_aggregate(h2, src_g, dst_g, zeros128)
    return _tc_final(agg2_p, deg_p, b2.reshape(1, OUT_DIM))


# speedup vs baseline: 9.8191x; 1.0033x over previous
import functools

import jax
import jax.numpy as jnp
from jax import lax
from jax.experimental import pallas as pl
from jax.experimental.pallas import tpu as pltpu
from jax.experimental.pallas import tpu_sc as plsc

N = 10000
E = 320000
IN_DIM = 128
HID_DIM = 256
OUT_DIM = 128

NC = 2
NS = 16
NW = NC * NS
RPT = 624
TAIL0 = RPT * NS
TAIL = N - TAIL0
GROUP = 125
N_GROUPS = E // GROUP
GROUPS_PER_TILE = N_GROUPS // NW
STAGE_A = 40
STAGE_B = GROUPS_PER_TILE - STAGE_A

_mesh = plsc.VectorSubcoreMesh(
    core_axis_name="c", subcore_axis_name="s", num_cores=NC, num_subcores=NS)


def _worker(c, s):
    return s * NC + c


def _zero_stripe(s, zeros_hbm, sh):
    pltpu.sync_copy(zeros_hbm.at[pl.ds(s * RPT, RPT)],
                    sh.at[pl.ds(s * RPT, RPT)])

    @pl.when(s == NS - 1)
    def _():
        pltpu.sync_copy(zeros_hbm.at[pl.ds(TAIL0, TAIL)],
                        sh.at[pl.ds(TAIL0, TAIL)])


def _publish_stripe(c, s, sh, out_hbm):
    pltpu.sync_copy(sh.at[pl.ds(s * RPT, RPT)],
                    out_hbm.at[c, pl.ds(s * RPT, RPT)])

    @pl.when(s == NS - 1)
    def _():
        pltpu.sync_copy(sh.at[pl.ds(TAIL0, TAIL)],
                        out_hbm.at[c, pl.ds(TAIL0, TAIL)])


def _sc_degrees_body(src_g_hbm, dst_g_hbm, ones_hbm, zeros_hbm, deg_hbm,
                     src_v, dst_v, ones_v, acc_sh, sem):
    c = lax.axis_index("c")
    s = lax.axis_index("s")
    _zero_stripe(s, zeros_hbm, acc_sh)
    w = _worker(c, s)
    pltpu.sync_copy(src_g_hbm.at[w], src_v)
    pltpu.sync_copy(dst_g_hbm.at[w], dst_v)
    plsc.subcore_barrier()

    K = 10

    def scatter_pass(idx_v):
        def chunk(k, carry):
            for j in range(K):
                pltpu.async_copy(ones_v, acc_sh.at[idx_v.at[k * K + j]],
                                 sem, add=True)
            for j in range(K):
                pltpu.make_async_copy(ones_v, acc_sh.at[idx_v.at[k * K + j]],
                                      sem).wait()
            return carry

        lax.fori_loop(0, GROUPS_PER_TILE // K, chunk, 0)

    pltpu.sync_copy(ones_hbm.at[0], ones_v)
    scatter_pass(src_v)
    pltpu.sync_copy(ones_hbm.at[1], ones_v)
    scatter_pass(dst_v)
    plsc.subcore_barrier()
    _publish_stripe(c, s, acc_sh, deg_hbm)


_sc_degrees = pl.kernel(
    _sc_degrees_body,
    out_type=jax.ShapeDtypeStruct((NC, N, IN_DIM), jnp.float32),
    mesh=_mesh,
    scratch_types=[
        pltpu.VMEM((GROUPS_PER_TILE, GROUP), jnp.int32),
        pltpu.VMEM((GROUPS_PER_TILE, GROUP), jnp.int32),
        pltpu.VMEM((GROUP, IN_DIM), jnp.float32),
        pltpu.VMEM_SHARED((N, IN_DIM), jnp.float32),
        pltpu.SemaphoreType.DMA,
    ],
)


def _sc_aggregate_body(y_hbm, src_g_hbm, dst_g_hbm, zeros_hbm, out_hbm,
                       src_v, dst_v, rows_a, rows_b, acc_sh, sem_a, sem_b):
    c = lax.axis_index("c")
    s = lax.axis_index("s")
    _zero_stripe(s, zeros_hbm, acc_sh)
    w = _worker(c, s)

    def stage(off, n):
        pltpu.sync_copy(src_g_hbm.at[w, pl.ds(off, n)], src_v.at[pl.ds(0, n)])
        pltpu.sync_copy(dst_g_hbm.at[w, pl.ds(off, n)], dst_v.at[pl.ds(0, n)])

    def span(n):
        pltpu.async_copy(y_hbm.at[src_v.at[0]], rows_a, sem_a)

        def body(i, carry):
            ga = 2 * i
            gb = 2 * i + 1
            pltpu.make_async_copy(y_hbm.at[src_v.at[ga]], rows_a, sem_a).wait()
            pltpu.async_copy(y_hbm.at[src_v.at[gb]], rows_b, sem_b)
            pltpu.sync_copy(rows_a, acc_sh.at[dst_v.at[ga]], add=True)
            pltpu.make_async_copy(y_hbm.at[src_v.at[gb]], rows_b, sem_b).wait()

            @pl.when(i < n // 2 - 1)
            def _():
                pltpu.async_copy(y_hbm.at[src_v.at[gb + 1]], rows_a, sem_a)

            pltpu.sync_copy(rows_b, acc_sh.at[dst_v.at[gb]], add=True)
            return carry

        lax.fori_loop(0, n // 2, body, 0)

    stage(0, STAGE_A)
    plsc.subcore_barrier()
    span(STAGE_A)
    stage(STAGE_A, STAGE_B)
    span(STAGE_B)
    plsc.subcore_barrier()
    _publish_stripe(c, s, acc_sh, out_hbm)


_sc_aggregate = pl.kernel(
    _sc_aggregate_body,
    out_type=jax.ShapeDtypeStruct((NC, N, IN_DIM), jnp.float32),
    mesh=_mesh,
    scratch_types=[
        pltpu.VMEM((STAGE_A, GROUP), jnp.int32),
        pltpu.VMEM((STAGE_A, GROUP), jnp.int32),
        pltpu.VMEM((GROUP, IN_DIM), jnp.float32),
        pltpu.VMEM((GROUP, IN_DIM), jnp.float32),
        pltpu.VMEM_SHARED((N, IN_DIM), jnp.float32),
        pltpu.SemaphoreType.DMA,
        pltpu.SemaphoreType.DMA,
    ],
)


RB = 1000


def _deg_scale(deg_ref, col):
    deg = deg_ref[0, :, col] + deg_ref[1, :, col]
    return lax.rsqrt(jnp.maximum(deg, 1.0))


def _tc_prep_body(deg_ref, x_ref, xs_ref, sc_ref):
    so = _deg_scale(deg_ref, 0)
    si = _deg_scale(deg_ref, 1)
    xs_ref[...] = x_ref[...] * so[:, None]
    sc_ref[...] = jnp.stack([si, so], axis=1)


def _tc_prep(deg_p, x):
    return pl.pallas_call(
        _tc_prep_body,
        grid=(N // RB,),
        in_specs=[
            pl.BlockSpec((NC, RB, IN_DIM), lambda i: (0, i, 0)),
            pl.BlockSpec((RB, IN_DIM), lambda i: (i, 0)),
        ],
        out_specs=[
            pl.BlockSpec((RB, IN_DIM), lambda i: (i, 0)),
            pl.BlockSpec((RB, 2), lambda i: (i, 0)),
        ],
        out_shape=[
            jax.ShapeDtypeStruct((N, IN_DIM), jnp.float32),
            jax.ShapeDtypeStruct((N, 2), jnp.float32),
        ],
    )(deg_p, x)


def _tc_dense_body(aggp_ref, sc_ref, w1_ref, b1_ref, w2_ref, h2_ref):
    agg = aggp_ref[0] + aggp_ref[1]
    si = sc_ref[:, 0]
    so = sc_ref[:, 1]
    t = jnp.dot(agg, w1_ref[...], preferred_element_type=jnp.float32)
    t = t * si[:, None] + b1_ref[...]
    t = jnp.maximum(t, 0.0) * so[:, None]
    h2_ref[...] = jnp.dot(t, w2_ref[...], preferred_element_type=jnp.float32)


def _tc_dense(agg1_p, scales, w1, b1, w2):
    return pl.pallas_call(
        _tc_dense_body,
        grid=(N // RB,),
        in_specs=[
            pl.BlockSpec((NC, RB, IN_DIM), lambda i: (0, i, 0)),
            pl.BlockSpec((RB, 2), lambda i: (i, 0)),
            pl.BlockSpec((IN_DIM, HID_DIM), lambda i: (0, 0)),
            pl.BlockSpec((1, HID_DIM), lambda i: (0, 0)),
            pl.BlockSpec((HID_DIM, OUT_DIM), lambda i: (0, 0)),
        ],
        out_specs=pl.BlockSpec((RB, OUT_DIM), lambda i: (i, 0)),
        out_shape=jax.ShapeDtypeStruct((N, OUT_DIM), jnp.float32),
    )(agg1_p, scales, w1, b1, w2)


def _tc_final_body(aggp_ref, sc_ref, b2_ref, out_ref):
    agg = aggp_ref[0] + aggp_ref[1]
    si = sc_ref[:, 0]
    out_ref[...] = agg * si[:, None] + b2_ref[...]


def _tc_final(agg2_p, scales, b2):
    return pl.pallas_call(
        _tc_final_body,
        grid=(N // RB,),
        in_specs=[
            pl.BlockSpec((NC, RB, OUT_DIM), lambda i: (0, i, 0)),
            pl.BlockSpec((RB, 2), lambda i: (i, 0)),
            pl.BlockSpec((1, OUT_DIM), lambda i: (0, 0)),
        ],
        out_specs=pl.BlockSpec((RB, OUT_DIM), lambda i: (i, 0)),
        out_shape=jax.ShapeDtypeStruct((N, OUT_DIM), jnp.float32),
    )(agg2_p, scales, b2)


def kernel(features, edge_index, W1, b1, W2, b2):
    src_g = edge_index[0].reshape(NW, GROUPS_PER_TILE, GROUP)
    dst_g = edge_index[1].reshape(NW, GROUPS_PER_TILE, GROUP)
    ones2 = jnp.zeros((2, GROUP, IN_DIM), jnp.float32)
    ones2 = ones2.at[0, :, 0].set(1.0).at[1, :, 1].set(1.0)
    zeros128 = jnp.zeros((N, IN_DIM), jnp.float32)

    deg_p = _sc_degrees(src_g, dst_g, ones2, zeros128)
    x_scaled, scales = _tc_prep(deg_p, features)
    agg1_p = _sc_aggregate(x_scaled, src_g, dst_g, zeros128)
    h2 = _tc_dense(agg1_p, scales, W1, b1.reshape(1, HID_DIM), W2)
    agg2_p = _sc_aggregate(h2, src_g, dst_g, zeros128)
    return _tc_final(agg2_p, scales, b2.reshape(1, OUT_DIM))


# speedup vs baseline: 9.9102x; 1.0093x over previous
import functools

import jax
import jax.numpy as jnp
from jax import lax
from jax.experimental import pallas as pl
from jax.experimental.pallas import tpu as pltpu
from jax.experimental.pallas import tpu_sc as plsc

N = 10000
E = 320000
IN_DIM = 128
HID_DIM = 256
OUT_DIM = 128

NC = 2
NS = 16
NW = NC * NS
RPT = 624
TAIL0 = RPT * NS
TAIL = N - TAIL0
GROUP = 125
N_GROUPS = E // GROUP
GROUPS_PER_TILE = N_GROUPS // NW
STAGE_A = 40
STAGE_B = GROUPS_PER_TILE - STAGE_A

_mesh = plsc.VectorSubcoreMesh(
    core_axis_name="c", subcore_axis_name="s", num_cores=NC, num_subcores=NS)


def _worker(c, s):
    return s * NC + c


def _zero_stripe(s, zeros_hbm, sh):
    pltpu.sync_copy(zeros_hbm.at[pl.ds(s * RPT, RPT)],
                    sh.at[pl.ds(s * RPT, RPT)])

    @pl.when(s == NS - 1)
    def _():
        pltpu.sync_copy(zeros_hbm.at[pl.ds(TAIL0, TAIL)],
                        sh.at[pl.ds(TAIL0, TAIL)])


def _publish_stripe(c, s, sh, out_hbm):
    pltpu.sync_copy(sh.at[pl.ds(s * RPT, RPT)],
                    out_hbm.at[c, pl.ds(s * RPT, RPT)])

    @pl.when(s == NS - 1)
    def _():
        pltpu.sync_copy(sh.at[pl.ds(TAIL0, TAIL)],
                        out_hbm.at[c, pl.ds(TAIL0, TAIL)])


def _sc_degrees_body(src_g_hbm, dst_g_hbm, ones_hbm, zeros_hbm, deg_hbm,
                     src_v, dst_v, ones_v, acc_sh, sem):
    c = lax.axis_index("c")
    s = lax.axis_index("s")
    _zero_stripe(s, zeros_hbm, acc_sh)
    w = _worker(c, s)
    pltpu.sync_copy(src_g_hbm.at[w], src_v)
    pltpu.sync_copy(dst_g_hbm.at[w], dst_v)
    plsc.subcore_barrier()

    K = 10

    def scatter_pass(idx_v):
        def chunk(k, carry):
            for j in range(K):
                pltpu.async_copy(ones_v, acc_sh.at[idx_v.at[k * K + j]],
                                 sem, add=True)
            for j in range(K):
                pltpu.make_async_copy(ones_v, acc_sh.at[idx_v.at[k * K + j]],
                                      sem).wait()
            return carry

        lax.fori_loop(0, GROUPS_PER_TILE // K, chunk, 0)

    pltpu.sync_copy(ones_hbm.at[0], ones_v)
    scatter_pass(src_v)
    pltpu.sync_copy(ones_hbm.at[1], ones_v)
    scatter_pass(dst_v)
    plsc.subcore_barrier()
    _publish_stripe(c, s, acc_sh, deg_hbm)


_sc_degrees = pl.kernel(
    _sc_degrees_body,
    out_type=jax.ShapeDtypeStruct((NC, N, IN_DIM), jnp.float32),
    mesh=_mesh,
    scratch_types=[
        pltpu.VMEM((GROUPS_PER_TILE, GROUP), jnp.int32),
        pltpu.VMEM((GROUPS_PER_TILE, GROUP), jnp.int32),
        pltpu.VMEM((GROUP, IN_DIM), jnp.float32),
        pltpu.VMEM_SHARED((N, IN_DIM), jnp.float32),
        pltpu.SemaphoreType.DMA,
    ],
)


def _sc_aggregate_body(y_hbm, src_g_hbm, dst_g_hbm, zeros_hbm, out_hbm,
                       src_v, dst_v, rows_a, rows_b, acc_sh, sem_a, sem_b):
    c = lax.axis_index("c")
    s = lax.axis_index("s")
    _zero_stripe(s, zeros_hbm, acc_sh)
    w = _worker(c, s)

    def stage(off, n):
        pltpu.sync_copy(src_g_hbm.at[w, pl.ds(off, n)], src_v.at[pl.ds(0, n)])
        pltpu.sync_copy(dst_g_hbm.at[w, pl.ds(off, n)], dst_v.at[pl.ds(0, n)])

    def span(n):
        pltpu.async_copy(y_hbm.at[src_v.at[0]], rows_a, sem_a)

        def body(i, carry):
            ga = 2 * i
            gb = 2 * i + 1
            pltpu.make_async_copy(y_hbm.at[src_v.at[ga]], rows_a, sem_a).wait()
            pltpu.async_copy(y_hbm.at[src_v.at[gb]], rows_b, sem_b)
            pltpu.sync_copy(rows_a, acc_sh.at[dst_v.at[ga]], add=True)
            pltpu.make_async_copy(y_hbm.at[src_v.at[gb]], rows_b, sem_b).wait()

            @pl.when(i < n // 2 - 1)
            def _():
                pltpu.async_copy(y_hbm.at[src_v.at[gb + 1]], rows_a, sem_a)

            pltpu.sync_copy(rows_b, acc_sh.at[dst_v.at[gb]], add=True)
            return carry

        lax.fori_loop(0, n // 2, body, 0)

    stage(0, STAGE_A)
    plsc.subcore_barrier()
    span(STAGE_A)
    stage(STAGE_A, STAGE_B)
    span(STAGE_B)
    plsc.subcore_barrier()
    _publish_stripe(c, s, acc_sh, out_hbm)


_sc_aggregate = pl.kernel(
    _sc_aggregate_body,
    out_type=jax.ShapeDtypeStruct((NC, N, IN_DIM), jnp.float32),
    mesh=_mesh,
    scratch_types=[
        pltpu.VMEM((STAGE_A, GROUP), jnp.int32),
        pltpu.VMEM((STAGE_A, GROUP), jnp.int32),
        pltpu.VMEM((GROUP, IN_DIM), jnp.float32),
        pltpu.VMEM((GROUP, IN_DIM), jnp.float32),
        pltpu.VMEM_SHARED((N, IN_DIM), jnp.float32),
        pltpu.SemaphoreType.DMA,
        pltpu.SemaphoreType.DMA,
    ],
)


RB = 2000


def _deg_scale(deg_ref, col):
    deg = deg_ref[0, :, col] + deg_ref[1, :, col]
    return lax.rsqrt(jnp.maximum(deg, 1.0))


def _tc_prep_body(deg_ref, x_ref, xs_ref, sc_ref):
    so = _deg_scale(deg_ref, 0)
    si = _deg_scale(deg_ref, 1)
    xs_ref[...] = x_ref[...] * so[:, None]
    sc_ref[...] = jnp.stack([si, so], axis=1)


def _tc_prep(deg_p, x):
    return pl.pallas_call(
        _tc_prep_body,
        grid=(N // RB,),
        in_specs=[
            pl.BlockSpec((NC, RB, IN_DIM), lambda i: (0, i, 0)),
            pl.BlockSpec((RB, IN_DIM), lambda i: (i, 0)),
        ],
        out_specs=[
            pl.BlockSpec((RB, IN_DIM), lambda i: (i, 0)),
            pl.BlockSpec((RB, 2), lambda i: (i, 0)),
        ],
        out_shape=[
            jax.ShapeDtypeStruct((N, IN_DIM), jnp.float32),
            jax.ShapeDtypeStruct((N, 2), jnp.float32),
        ],
    )(deg_p, x)


def _tc_dense_body(aggp_ref, sc_ref, w1_ref, b1_ref, w2_ref, h2_ref):
    agg = aggp_ref[0] + aggp_ref[1]
    si = sc_ref[:, 0]
    so = sc_ref[:, 1]
    t = jnp.dot(agg, w1_ref[...], preferred_element_type=jnp.float32)
    t = t * si[:, None] + b1_ref[...]
    t = jnp.maximum(t, 0.0) * so[:, None]
    h2_ref[...] = jnp.dot(t, w2_ref[...], preferred_element_type=jnp.float32)


def _tc_dense(agg1_p, scales, w1, b1, w2):
    return pl.pallas_call(
        _tc_dense_body,
        grid=(N // RB,),
        in_specs=[
            pl.BlockSpec((NC, RB, IN_DIM), lambda i: (0, i, 0)),
            pl.BlockSpec((RB, 2), lambda i: (i, 0)),
            pl.BlockSpec((IN_DIM, HID_DIM), lambda i: (0, 0)),
            pl.BlockSpec((1, HID_DIM), lambda i: (0, 0)),
            pl.BlockSpec((HID_DIM, OUT_DIM), lambda i: (0, 0)),
        ],
        out_specs=pl.BlockSpec((RB, OUT_DIM), lambda i: (i, 0)),
        out_shape=jax.ShapeDtypeStruct((N, OUT_DIM), jnp.float32),
    )(agg1_p, scales, w1, b1, w2)


def _tc_final_body(aggp_ref, sc_ref, b2_ref, out_ref):
    agg = aggp_ref[0] + aggp_ref[1]
    si = sc_ref[:, 0]
    out_ref[...] = agg * si[:, None] + b2_ref[...]


def _tc_final(agg2_p, scales, b2):
    return pl.pallas_call(
        _tc_final_body,
        grid=(N // RB,),
        in_specs=[
            pl.BlockSpec((NC, RB, OUT_DIM), lambda i: (0, i, 0)),
            pl.BlockSpec((RB, 2), lambda i: (i, 0)),
            pl.BlockSpec((1, OUT_DIM), lambda i: (0, 0)),
        ],
        out_specs=pl.BlockSpec((RB, OUT_DIM), lambda i: (i, 0)),
        out_shape=jax.ShapeDtypeStruct((N, OUT_DIM), jnp.float32),
    )(agg2_p, scales, b2)


def kernel(features, edge_index, W1, b1, W2, b2):
    src_g = edge_index[0].reshape(NW, GROUPS_PER_TILE, GROUP)
    dst_g = edge_index[1].reshape(NW, GROUPS_PER_TILE, GROUP)
    ones2 = jnp.zeros((2, GROUP, IN_DIM), jnp.float32)
    ones2 = ones2.at[0, :, 0].set(1.0).at[1, :, 1].set(1.0)
    zeros128 = jnp.zeros((N, IN_DIM), jnp.float32)

    deg_p = _sc_degrees(src_g, dst_g, ones2, zeros128)
    x_scaled, scales = _tc_prep(deg_p, features)
    agg1_p = _sc_aggregate(x_scaled, src_g, dst_g, zeros128)
    h2 = _tc_dense(agg1_p, scales, W1, b1.reshape(1, HID_DIM), W2)
    agg2_p = _sc_aggregate(h2, src_g, dst_g, zeros128)
    return _tc_final(agg2_p, scales, b2.reshape(1, OUT_DIM))


# speedup vs baseline: 9.9153x; 1.0005x over previous
import jax
import jax.numpy as jnp
from jax import lax
from jax.experimental import pallas as pl
from jax.experimental.pallas import tpu as pltpu
from jax.experimental.pallas import tpu_sc as plsc

N = 10000
E = 320000
IN_DIM = 128
HID_DIM = 256
OUT_DIM = 128

NC = 2
NS = 16
NW = NC * NS
RPT = 624
TAIL0 = RPT * NS
TAIL = N - TAIL0
GROUP = 125
N_GROUPS = E // GROUP
GROUPS_PER_TILE = N_GROUPS // NW
STAGE_A = 40
STAGE_B = GROUPS_PER_TILE - STAGE_A

_mesh = plsc.VectorSubcoreMesh(
    core_axis_name="c", subcore_axis_name="s", num_cores=NC, num_subcores=NS)


def _worker(c, s):
    return s * NC + c


def _zero_stripe(s, zeros_hbm, sh):
    pltpu.sync_copy(zeros_hbm.at[pl.ds(s * RPT, RPT)],
                    sh.at[pl.ds(s * RPT, RPT)])

    @pl.when(s == NS - 1)
    def _():
        pltpu.sync_copy(zeros_hbm.at[pl.ds(TAIL0, TAIL)],
                        sh.at[pl.ds(TAIL0, TAIL)])


def _publish_stripe(c, s, sh, out_hbm):
    pltpu.sync_copy(sh.at[pl.ds(s * RPT, RPT)],
                    out_hbm.at[c, pl.ds(s * RPT, RPT)])

    @pl.when(s == NS - 1)
    def _():
        pltpu.sync_copy(sh.at[pl.ds(TAIL0, TAIL)],
                        out_hbm.at[c, pl.ds(TAIL0, TAIL)])


def _sc_degrees_body(src_g_hbm, dst_g_hbm, ones_hbm, zeros_hbm, deg_hbm,
                     src_v, dst_v, ones_v, acc_sh, sem):
    c = lax.axis_index("c")
    s = lax.axis_index("s")
    _zero_stripe(s, zeros_hbm, acc_sh)
    w = _worker(c, s)
    pltpu.sync_copy(src_g_hbm.at[w], src_v)
    pltpu.sync_copy(dst_g_hbm.at[w], dst_v)
    plsc.subcore_barrier()

    K = 10

    def scatter_pass(idx_v):
        def chunk(k, carry):
            for j in range(K):
                pltpu.async_copy(ones_v, acc_sh.at[idx_v.at[k * K + j]],
                                 sem, add=True)
            for j in range(K):
                pltpu.make_async_copy(ones_v, acc_sh.at[idx_v.at[k * K + j]],
                                      sem).wait()
            return carry

        lax.fori_loop(0, GROUPS_PER_TILE // K, chunk, 0)

    pltpu.sync_copy(ones_hbm.at[0], ones_v)
    scatter_pass(src_v)
    pltpu.sync_copy(ones_hbm.at[1], ones_v)
    scatter_pass(dst_v)
    plsc.subcore_barrier()
    _publish_stripe(c, s, acc_sh, deg_hbm)


_sc_degrees = pl.kernel(
    _sc_degrees_body,
    out_type=jax.ShapeDtypeStruct((NC, N, IN_DIM), jnp.float32),
    mesh=_mesh,
    scratch_types=[
        pltpu.VMEM((GROUPS_PER_TILE, GROUP), jnp.int32),
        pltpu.VMEM((GROUPS_PER_TILE, GROUP), jnp.int32),
        pltpu.VMEM((GROUP, IN_DIM), jnp.float32),
        pltpu.VMEM_SHARED((N, IN_DIM), jnp.float32),
        pltpu.SemaphoreType.DMA,
    ],
)


def _sc_aggregate_body(y_hbm, src_g_hbm, dst_g_hbm, zeros_hbm, out_hbm,
                       src_v, dst_v, rows_a, rows_b, acc_sh, sem_a, sem_b):
    c = lax.axis_index("c")
    s = lax.axis_index("s")
    _zero_stripe(s, zeros_hbm, acc_sh)
    w = _worker(c, s)

    def stage(off, n):
        pltpu.sync_copy(src_g_hbm.at[w, pl.ds(off, n)], src_v.at[pl.ds(0, n)])
        pltpu.sync_copy(dst_g_hbm.at[w, pl.ds(off, n)], dst_v.at[pl.ds(0, n)])

    def span(n):
        pltpu.async_copy(y_hbm.at[src_v.at[0]], rows_a, sem_a)

        def body(i, carry):
            ga = 2 * i
            gb = 2 * i + 1
            pltpu.make_async_copy(y_hbm.at[src_v.at[ga]], rows_a, sem_a).wait()
            pltpu.async_copy(y_hbm.at[src_v.at[gb]], rows_b, sem_b)
            pltpu.sync_copy(rows_a, acc_sh.at[dst_v.at[ga]], add=True)
            pltpu.make_async_copy(y_hbm.at[src_v.at[gb]], rows_b, sem_b).wait()

            @pl.when(i < n // 2 - 1)
            def _():
                pltpu.async_copy(y_hbm.at[src_v.at[gb + 1]], rows_a, sem_a)

            pltpu.sync_copy(rows_b, acc_sh.at[dst_v.at[gb]], add=True)
            return carry

        lax.fori_loop(0, n // 2, body, 0)

    stage(0, STAGE_A)
    plsc.subcore_barrier()
    span(STAGE_A)
    stage(STAGE_A, STAGE_B)
    span(STAGE_B)
    plsc.subcore_barrier()
    _publish_stripe(c, s, acc_sh, out_hbm)


_sc_aggregate = pl.kernel(
    _sc_aggregate_body,
    out_type=jax.ShapeDtypeStruct((NC, N, IN_DIM), jnp.float32),
    mesh=_mesh,
    scratch_types=[
        pltpu.VMEM((STAGE_A, GROUP), jnp.int32),
        pltpu.VMEM((STAGE_A, GROUP), jnp.int32),
        pltpu.VMEM((GROUP, IN_DIM), jnp.float32),
        pltpu.VMEM((GROUP, IN_DIM), jnp.float32),
        pltpu.VMEM_SHARED((N, IN_DIM), jnp.float32),
        pltpu.SemaphoreType.DMA,
        pltpu.SemaphoreType.DMA,
    ],
)


RB = 2000


def _deg_scale(deg_ref, col):
    deg = deg_ref[0, :, col] + deg_ref[1, :, col]
    return lax.rsqrt(jnp.maximum(deg, 1.0))


def _tc_prep_body(deg_ref, x_ref, xs_ref, sc_ref):
    so = _deg_scale(deg_ref, 0)
    si = _deg_scale(deg_ref, 1)
    xs_ref[...] = x_ref[...] * so[:, None]
    sc_ref[...] = jnp.stack([si, so], axis=1)


def _tc_prep(deg_p, x):
    return pl.pallas_call(
        _tc_prep_body,
        grid=(N // RB,),
        in_specs=[
            pl.BlockSpec((NC, RB, IN_DIM), lambda i: (0, i, 0)),
            pl.BlockSpec((RB, IN_DIM), lambda i: (i, 0)),
        ],
        out_specs=[
            pl.BlockSpec((RB, IN_DIM), lambda i: (i, 0)),
            pl.BlockSpec((RB, 2), lambda i: (i, 0)),
        ],
        out_shape=[
            jax.ShapeDtypeStruct((N, IN_DIM), jnp.float32),
            jax.ShapeDtypeStruct((N, 2), jnp.float32),
        ],
    )(deg_p, x)


def _tc_dense_body(aggp_ref, sc_ref, w1_ref, b1_ref, w2_ref, h2_ref):
    agg = aggp_ref[0] + aggp_ref[1]
    si = sc_ref[:, 0]
    so = sc_ref[:, 1]
    t = jnp.dot(agg, w1_ref[...], preferred_element_type=jnp.float32)
    t = t * si[:, None] + b1_ref[...]
    t = jnp.maximum(t, 0.0) * so[:, None]
    h2_ref[...] = jnp.dot(t, w2_ref[...], preferred_element_type=jnp.float32)


def _tc_dense(agg1_p, scales, w1, b1, w2):
    return pl.pallas_call(
        _tc_dense_body,
        grid=(N // RB,),
        in_specs=[
            pl.BlockSpec((NC, RB, IN_DIM), lambda i: (0, i, 0)),
            pl.BlockSpec((RB, 2), lambda i: (i, 0)),
            pl.BlockSpec((IN_DIM, HID_DIM), lambda i: (0, 0)),
            pl.BlockSpec((1, HID_DIM), lambda i: (0, 0)),
            pl.BlockSpec((HID_DIM, OUT_DIM), lambda i: (0, 0)),
        ],
        out_specs=pl.BlockSpec((RB, OUT_DIM), lambda i: (i, 0)),
        out_shape=jax.ShapeDtypeStruct((N, OUT_DIM), jnp.float32),
    )(agg1_p, scales, w1, b1, w2)


def _tc_final_body(aggp_ref, sc_ref, b2_ref, out_ref):
    agg = aggp_ref[0] + aggp_ref[1]
    si = sc_ref[:, 0]
    out_ref[...] = agg * si[:, None] + b2_ref[...]


def _tc_final(agg2_p, scales, b2):
    return pl.pallas_call(
        _tc_final_body,
        grid=(N // RB,),
        in_specs=[
            pl.BlockSpec((NC, RB, OUT_DIM), lambda i: (0, i, 0)),
            pl.BlockSpec((RB, 2), lambda i: (i, 0)),
            pl.BlockSpec((1, OUT_DIM), lambda i: (0, 0)),
        ],
        out_specs=pl.BlockSpec((RB, OUT_DIM), lambda i: (i, 0)),
        out_shape=jax.ShapeDtypeStruct((N, OUT_DIM), jnp.float32),
    )(agg2_p, scales, b2)


def kernel(features, edge_index, W1, b1, W2, b2):
    src_g = edge_index[0].reshape(NW, GROUPS_PER_TILE, GROUP)
    dst_g = edge_index[1].reshape(NW, GROUPS_PER_TILE, GROUP)
    ones2 = jnp.zeros((2, GROUP, IN_DIM), jnp.float32)
    ones2 = ones2.at[0, :, 0].set(1.0).at[1, :, 1].set(1.0)
    zeros128 = jnp.zeros((N, IN_DIM), jnp.float32)

    deg_p = _sc_degrees(src_g, dst_g, ones2, zeros128)
    x_scaled, scales = _tc_prep(deg_p, features)
    agg1_p = _sc_aggregate(x_scaled, src_g, dst_g, zeros128)
    h2 = _tc_dense(agg1_p, scales, W1, b1.reshape(1, HID_DIM), W2)
    agg2_p = _sc_aggregate(h2, src_g, dst_g, zeros128)
    return _tc_final(agg2_p, scales, b2.reshape(1, OUT_DIM))
